# baseline (speedup 1.0000x reference)
import jax
import jax.numpy as jnp
from jax import lax
from jax.experimental import pallas as pl
from jax.experimental.pallas import tpu as pltpu
from jax.experimental.pallas import tpu_sc as plsc

B = 16384
N_DENSE = 13
N_SPARSE = 26
VOCAB = 100000
D = 16
EPS = 1e-5

NW = 32
ROWS_PW = B // NW
GROUP = 64
GROUPS_PW = ROWS_PW // GROUP
IDX_PER_GROUP = GROUP * N_SPARSE
CHUNKS_PER_GROUP = IDX_PER_GROUP // 128
XS_VECS_PW = ROWS_PW * N_SPARSE // 16
IDX_ROWS_PW = ROWS_PW * N_SPARSE // 128


def _sc_body(xs_hbm, emb_hbm, out_hbm,
             xs_v, idx_v, buf0, buf1, cross_v, sem0, sem1):
    cid = lax.axis_index("c")
    sid = lax.axis_index("s")
    wid = sid * 2 + cid

    pltpu.sync_copy(xs_hbm.at[pl.ds(wid * XS_VECS_PW, XS_VECS_PW)], xs_v)

    lanes = lax.iota(jnp.int32, 16)

    def ibody(j, _):
        v = xs_v[j]
        p = (wid * XS_VECS_PW + j) * 16 + lanes
        f = p % N_SPARSE
        idx = v.astype(jnp.int32) + f * VOCAB
        idx_v[j // 8, pl.ds((j % 8) * 16, 16)] = idx
        return 0

    lax.fori_loop(0, XS_VECS_PW, ibody, 0)

    bufs = (buf0, buf1)
    sems = (sem0, sem1)

    def fire(g, buf, sem):
        copies = []
        for c in range(CHUNKS_PER_GROUP):
            chunk = g * CHUNKS_PER_GROUP + c
            copies.append(
                pltpu.async_copy(emb_hbm.at[idx_v.at[chunk]],
                                 buf.at[pl.ds(c * 128, 128)], sem))
        return copies

    def compute(g, buf):
        def rbody(r, _):
            base = r * N_SPARSE
            v = buf[base]
            s = v
            q = v * v
            for f in range(1, N_SPARSE):
                v = buf[base + f]
                s = s + v
                q = q + v * v
            cross_v[g * GROUP + r] = 0.5 * (s * s - q)
            return 0

        lax.fori_loop(0, GROUP, rbody, 0)

    pending = [None, None]
    pending[0] = fire(0, bufs[0], sems[0])
    for g in range(GROUPS_PW):
        nxt = g + 1
        if nxt < GROUPS_PW:
            pending[nxt % 2] = fire(nxt, bufs[nxt % 2], sems[nxt % 2])
        for cp in pending[g % 2]:
            cp.wait()
        compute(g, bufs[g % 2])

    pltpu.sync_copy(cross_v, out_hbm.at[pl.ds(wid * ROWS_PW, ROWS_PW)])


def _sc_bi_interaction(xs, embf):
    mesh = plsc.VectorSubcoreMesh(core_axis_name="c", subcore_axis_name="s")
    kern = pl.kernel(
        _sc_body,
        out_type=jax.ShapeDtypeStruct((B, D), jnp.float32),
        mesh=mesh,
        scratch_types=[
            pltpu.VMEM((XS_VECS_PW, 16), jnp.float32),
            pltpu.VMEM((IDX_ROWS_PW, 128), jnp.int32),
            pltpu.VMEM((IDX_PER_GROUP, D), jnp.float32),
            pltpu.VMEM((IDX_PER_GROUP, D), jnp.float32),
            pltpu.VMEM((ROWS_PW, D), jnp.float32),
            pltpu.SemaphoreType.DMA,
            pltpu.SemaphoreType.DMA,
        ],
        compiler_params=pltpu.CompilerParams(use_tc_tiling_on_sc=False),
    )
    return kern(xs, embf)


def _tc_body(x_ref, cr_ref, gd_ref, gc_ref, bd_ref, bc_ref,
             w1d_ref, w1c_ref, b1_ref, w2_ref, b2_ref, w3_ref, b3_ref,
             o_ref):
    dense = x_ref[:, 0:N_DENSE]
    cross = cr_ref[:]

    mu_d = jnp.mean(dense, axis=0, keepdims=True)
    cd = dense - mu_d
    var_d = jnp.mean(cd * cd, axis=0, keepdims=True)
    zd = cd * lax.rsqrt(var_d + EPS) * gd_ref[:][None, :] + bd_ref[:][None, :]

    mu_c = jnp.mean(cross, axis=0, keepdims=True)
    cc = cross - mu_c
    var_c = jnp.mean(cc * cc, axis=0, keepdims=True)
    zc = cc * lax.rsqrt(var_c + EPS) * gc_ref[:][None, :] + bc_ref[:][None, :]

    h = (jnp.dot(zd, w1d_ref[:], preferred_element_type=jnp.float32)
         + jnp.dot(zc, w1c_ref[:], preferred_element_type=jnp.float32)
         + b1_ref[:][None, :])
    h = jnp.maximum(h, 0.0)
    h = jnp.maximum(
        jnp.dot(h, w2_ref[:], preferred_element_type=jnp.float32)
        + b2_ref[:][None, :], 0.0)
    t = (jnp.dot(h, w3_ref[:], preferred_element_type=jnp.float32)
         + b3_ref[:][None, :])
    o_ref[:] = 1.0 / (1.0 + jnp.exp(-t))


def _tc_bn_mlp(x, cross, gamma, beta, W1, b1, W2, b2, W3, b3):
    return pl.pallas_call(
        _tc_body,
        out_shape=jax.ShapeDtypeStruct((B, 1), jnp.float32),
    )(x, cross, gamma[:N_DENSE], gamma[N_DENSE:], beta[:N_DENSE],
      beta[N_DENSE:], W1[:N_DENSE], W1[N_DENSE:], b1, W2, b2, W3, b3)


def kernel(x, emb, gamma, beta, W1, b1, W2, b2, W3, b3):
    xs = x[:, N_DENSE:].reshape(B * N_SPARSE // 16, 16)
    embf = emb.reshape(N_SPARSE * VOCAB, D)
    cross = _sc_bi_interaction(xs, embf)
    out = _tc_bn_mlp(x, cross, gamma, beta, W1, b1, W2, b2, W3, b3)
    return out[:, 0]
